# baseline (speedup 1.0000x reference)
import functools

import jax
import jax.numpy as jnp
from jax import lax
from jax.experimental import pallas as pl
from jax.experimental.pallas import tpu as pltpu
from jax.experimental.pallas import tpu_sc as plsc

ND = 10000
NP = 40000
N = ND + NP
E = 800000
LBL = 100000
DIN = 128
H = 64
HH = 32

NC, NS, LANES = 2, 16, 16
EP = 802816
EPT_SCAT = EP // NS
EPT_DEG = EP // (NC * NS)
GROUP = 128
DR2 = 65536
ACC_ROWS = N + LANES
RPT = N // NS
LP = 100352
LPT = LP // (NC * NS)
BROW = 1000

_mesh = plsc.VectorSubcoreMesh(
    core_axis_name="c", subcore_axis_name="s", num_cores=NC, num_subcores=NS)
_sc_params = pltpu.CompilerParams(
    needs_layout_passes=False, use_tc_tiling_on_sc=False)

f32 = jnp.float32
i32 = jnp.int32


@functools.partial(
    pl.kernel,
    out_type=jax.ShapeDtypeStruct((DR2 // GROUP, GROUP), f32),
    mesh=_mesh,
    scratch_types=[
        pltpu.VMEM((DR2 // GROUP, GROUP), f32),
        pltpu.VMEM((14, GROUP), i32),
        pltpu.VMEM((4, GROUP), i32),
        pltpu.VMEM_SHARED((DR2 // GROUP, GROUP), f32),
    ],
    compiler_params=_sc_params,
)
def _sc_degree(dst_hbm, deg_hbm, dloc, dbuf, ibuf, spacc):
    c = lax.axis_index("c")
    s = lax.axis_index("s")
    wid = c * NS + s
    zero16 = jnp.zeros((LANES,), f32)
    ones16 = jnp.ones((LANES,), f32)
    iota16 = jnp.arange(LANES, dtype=i32)

    def zbody(i, _):
        for k in range(GROUP // LANES):
            dloc[i, pl.ds(k * LANES, LANES)] = zero16
        return 0
    lax.fori_loop(0, DR2 // GROUP, zbody, 0)

    rbase = s * (EPT_SCAT // GROUP)

    def ebody(g, _):
        pltpu.sync_copy(dst_hbm.at[pl.ds(rbase + g * 14, 14)], dbuf)
        for k in range(14):
            for t in range(GROUP // LANES):
                idx = dbuf[k, pl.ds(t * LANES, LANES)]
                plsc.addupdate_scatter(dloc, [idx >> 7, idx & 127], ones16)
        return 0
    lax.fori_loop(0, EPT_SCAT // GROUP // 14, ebody, 0)

    @pl.when(s == 0)
    def _():
        pltpu.sync_copy(dloc, spacc)
    for ci in range(4):
        for k in range(GROUP // LANES):
            ibuf[ci, pl.ds(k * LANES, LANES)] = iota16 + (ci * GROUP + k * LANES)
    plsc.subcore_barrier()

    @pl.when(s != 0)
    def _():
        for ci in range(4):
            pltpu.sync_copy(dloc.at[pl.ds(ci * GROUP, GROUP)],
                            spacc.at[ibuf.at[ci]], add=True)
    plsc.subcore_barrier()

    pltpu.sync_copy(spacc.at[pl.ds(16 * wid, 16)],
                    deg_hbm.at[pl.ds(16 * wid, 16)])


@functools.partial(
    pl.kernel,
    out_type=jax.ShapeDtypeStruct((NC * N, HH), f32),
    mesh=_mesh,
    scratch_types=[
        pltpu.VMEM_SHARED((ACC_ROWS, HH), f32),
        pltpu.VMEM((8, GROUP), i32),
        pltpu.VMEM((8, GROUP), i32),
        pltpu.VMEM((4, GROUP, HH), f32),
        pltpu.SemaphoreType.DMA((4,)),
        pltpu.SemaphoreType.DMA((4,)),
        pltpu.SemaphoreType.DMA((2,)),
    ],
    compiler_params=_sc_params,
)
def _sc_scatter(z_hbm, src_hbm, dst_hbm, acc_hbm,
                acc_sh, sbuf, dbuf, rows, gsems, ssems, isems):
    c = lax.axis_index("c")
    s = lax.axis_index("s")
    rbase = s * RPT
    pltpu.sync_copy(z_hbm.at[pl.ds(c * N + rbase, RPT)],
                    acc_sh.at[pl.ds(rbase, RPT)])
    plsc.subcore_barrier()

    offv = jnp.full((LANES,), c * N, i32)
    row0 = s * (EPT_SCAT // GROUP)

    def _drain(b):
        pltpu.make_async_copy(z_hbm.at[pl.ds(0, GROUP)], rows.at[b],
                              ssems.at[b]).wait()

    def gbody(g, _):
        r0 = row0 + g * 8
        ids = pltpu.async_copy(src_hbm.at[pl.ds(r0, 8)], sbuf, isems.at[0])
        idd = pltpu.async_copy(dst_hbm.at[pl.ds(r0, 8)], dbuf, isems.at[1])

        @pl.when(g > 0)
        def _():
            for b in range(4):
                _drain(b)

        ids.wait()
        idd.wait()
        for k in range(8):
            for t in range(GROUP // LANES):
                sbuf[k, pl.ds(t * LANES, LANES)] = (
                    sbuf[k, pl.ds(t * LANES, LANES)] + offv)
        gds = [pltpu.async_copy(z_hbm.at[sbuf.at[b]], rows.at[b], gsems.at[b])
               for b in range(4)]
        sds = []
        for b in range(4):
            gds[b].wait()
            sds.append(pltpu.async_copy(
                rows.at[b], acc_sh.at[dbuf.at[b]], ssems.at[b], add=True))
        gds2 = []
        for b in range(4):
            sds[b].wait()
            gds2.append(pltpu.async_copy(z_hbm.at[sbuf.at[4 + b]],
                                         rows.at[b], gsems.at[b]))
        for b in range(4):
            gds2[b].wait()
            pltpu.async_copy(
                rows.at[b], acc_sh.at[dbuf.at[4 + b]], ssems.at[b], add=True)
        return 0
    lax.fori_loop(0, EPT_SCAT // GROUP // 8, gbody, 0)

    for b in range(4):
        _drain(b)

    plsc.subcore_barrier()
    pltpu.sync_copy(acc_sh.at[pl.ds(rbase, RPT)],
                    acc_hbm.at[pl.ds(c * N + rbase, RPT)])


@functools.partial(
    pl.kernel,
    out_type=jax.ShapeDtypeStruct((LP,), f32),
    mesh=_mesh,
    scratch_types=[
        pltpu.VMEM((N,), f32),
        pltpu.VMEM((LPT,), i32),
        pltpu.VMEM((LPT,), i32),
        pltpu.VMEM((LPT,), f32),
    ],
    compiler_params=_sc_params,
)
def _sc_score(p_hbm, i0_hbm, i1_hbm, out_hbm, pbuf, i0, i1, ob):
    c = lax.axis_index("c")
    s = lax.axis_index("s")
    wid = c * NS + s
    base = wid * LPT
    pltpu.sync_copy(p_hbm, pbuf)
    pltpu.sync_copy(i0_hbm.at[pl.ds(base, LPT)], i0)
    pltpu.sync_copy(i1_hbm.at[pl.ds(base, LPT)], i1)

    def sbody(j, _):
        a = plsc.load_gather(pbuf, [i0[pl.ds(j * LANES, LANES)]])
        b = plsc.load_gather(pbuf, [i1[pl.ds(j * LANES, LANES)]])
        t = a * b
        ob[pl.ds(j * LANES, LANES)] = 1.0 / (1.0 + jnp.exp(-t))
        return 0
    lax.fori_loop(0, LPT // LANES, sbody, 0)

    pltpu.sync_copy(ob, out_hbm.at[pl.ds(base, LPT)])


def _dinv_block(deg_ref):
    return lax.rsqrt(deg_ref[...] + 1.0)


def _enc_body(xd_ref, xp_ref, Wd_ref, bd_ref, Wp_ref, bp_ref, W0_ref,
              degp_ref, z_ref):
    is_d = pl.program_id(0) < ND // BROW
    x = jnp.where(is_d, xd_ref[...], xp_ref[...])
    W = jnp.where(is_d, Wd_ref[...], Wp_ref[...])
    b = jnp.where(is_d, bd_ref[...], bp_ref[...])
    x1 = jnp.maximum(x @ W + b, 0.0)
    h = x1 @ W0_ref[...]
    d = _dinv_block(degp_ref)
    z_ref[0] = h[:, :HH] * d
    z_ref[1] = h[:, HH:] * d


def _layer_body(acc_ref, degp_ref, b_ref, W_ref, z_ref):
    d = _dinv_block(degp_ref)
    x = jnp.concatenate([acc_ref[0] * d, acc_ref[1] * d], axis=1) + b_ref[...]
    x = jnp.maximum(x, 0.0)
    y = x @ W_ref[...]
    z_ref[0] = y[:, :HH] * d
    z_ref[1] = y[:, HH:] * d


def _final_body(acc_ref, degp_ref, b_ref, lw_ref, lb_ref, p_ref):
    d = _dinv_block(degp_ref)
    x = jnp.concatenate([acc_ref[0] * d, acc_ref[1] * d], axis=1) + b_ref[...]
    x = jnp.maximum(x, 0.0)
    p_ref[...] = x @ lw_ref[...] + lb_ref[0, 0]


def _full(shape):
    return pl.BlockSpec(shape, lambda i: tuple(0 for _ in shape))


_GRID = N // BROW

_enc_call = pl.pallas_call(
    _enc_body,
    grid=(_GRID,),
    in_specs=[
        pl.BlockSpec((BROW, DIN), lambda i: (jnp.minimum(i, ND // BROW - 1), 0)),
        pl.BlockSpec((BROW, DIN), lambda i: (jnp.maximum(i - ND // BROW, 0), 0)),
        _full((DIN, H)), _full((1, H)), _full((DIN, H)), _full((1, H)),
        _full((H, H)),
        pl.BlockSpec((BROW, 1), lambda i: (i, 0)),
    ],
    out_specs=pl.BlockSpec((2, BROW, HH), lambda i: (0, i, 0)),
    out_shape=jax.ShapeDtypeStruct((2, N, HH), f32),
)

_DCB = 6400

_layer_call = pl.pallas_call(
    _layer_body,
    grid=(_GRID,),
    in_specs=[
        pl.BlockSpec((2, BROW, HH), lambda i: (0, i, 0)),
        pl.BlockSpec((BROW, 1), lambda i: (i, 0)),
        _full((1, H)), _full((H, H)),
    ],
    out_specs=pl.BlockSpec((2, BROW, HH), lambda i: (0, i, 0)),
    out_shape=jax.ShapeDtypeStruct((2, N, HH), f32),
)

_final_call = pl.pallas_call(
    _final_body,
    grid=(_GRID,),
    in_specs=[
        pl.BlockSpec((2, BROW, HH), lambda i: (0, i, 0)),
        pl.BlockSpec((BROW, 1), lambda i: (i, 0)),
        _full((1, H)), _full((H, 1)), _full((1, 1)),
    ],
    out_specs=pl.BlockSpec((BROW, 1), lambda i: (i, 0)),
    out_shape=jax.ShapeDtypeStruct((N, 1), f32),
)


def kernel(x_drug, x_protein, edge_index, edge_label_index,
           W_drug, b_drug, W_prot, b_prot,
           conv_W0, conv_b0, conv_W1, conv_b1, conv_W2, conv_b2,
           lin_W, lin_b):
    npad = EP - E
    src = jnp.concatenate(
        [edge_index[0], jnp.zeros((npad,), i32)]).reshape(EP // GROUP, GROUP)
    dst = jnp.concatenate(
        [edge_index[1], jnp.full((npad,), N, i32)]).reshape(EP // GROUP, GROUP)

    degp = _sc_degree(dst).reshape(DR2, 1)[:N]

    z = _enc_call(x_drug, x_protein, W_drug, b_drug.reshape(1, H),
                  W_prot, b_prot.reshape(1, H), conv_W0,
                  degp).reshape(NC * N, HH)
    for Wb in ((conv_b0, conv_W1), (conv_b1, conv_W2)):
        acc = _sc_scatter(z, src, dst).reshape(2, N, HH)
        z = _layer_call(acc, degp, Wb[0].reshape(1, H), Wb[1]).reshape(NC * N, HH)

    acc = _sc_scatter(z, src, dst).reshape(2, N, HH)
    p = _final_call(acc, degp, conv_b2.reshape(1, H),
                    lin_W, lin_b.reshape(1, 1)).reshape(N)

    lpad = LP - LBL
    i0 = jnp.concatenate([edge_label_index[0], jnp.zeros((lpad,), i32)])
    i1 = jnp.concatenate([edge_label_index[1], jnp.zeros((lpad,), i32)])
    return _sc_score(p, i0, i1)[:LBL]


# speedup vs baseline: 1.0775x; 1.0775x over previous
import functools

import jax
import jax.numpy as jnp
from jax import lax
from jax.experimental import pallas as pl
from jax.experimental.pallas import tpu as pltpu
from jax.experimental.pallas import tpu_sc as plsc

ND = 10000
NP = 40000
N = ND + NP
E = 800000
LBL = 100000
DIN = 128
H = 64
HH = 32

NC, NS, LANES = 2, 16, 16
EP = 802816
EPT_SCAT = EP // NS
EPT_DEG = EP // (NC * NS)
GROUP = 128
DR2 = 65536
ACC_ROWS = N + LANES
RPT = N // NS
LP = 100352
LPT = LP // (NC * NS)
BROW = 2000

_mesh = plsc.VectorSubcoreMesh(
    core_axis_name="c", subcore_axis_name="s", num_cores=NC, num_subcores=NS)
_sc_params = pltpu.CompilerParams(
    needs_layout_passes=False, use_tc_tiling_on_sc=False)

f32 = jnp.float32
i32 = jnp.int32


@functools.partial(
    pl.kernel,
    out_type=jax.ShapeDtypeStruct((DR2 // GROUP, GROUP), f32),
    mesh=_mesh,
    scratch_types=[
        pltpu.VMEM((DR2 // GROUP, GROUP), f32),
        pltpu.VMEM((14, GROUP), i32),
        pltpu.VMEM((4, GROUP), i32),
        pltpu.VMEM_SHARED((DR2 // GROUP, GROUP), f32),
    ],
    compiler_params=_sc_params,
)
def _sc_degree(dst_hbm, deg_hbm, dloc, dbuf, ibuf, spacc):
    c = lax.axis_index("c")
    s = lax.axis_index("s")
    wid = c * NS + s
    zero16 = jnp.zeros((LANES,), f32)
    ones16 = jnp.ones((LANES,), f32)
    iota16 = jnp.arange(LANES, dtype=i32)

    def zbody(i, _):
        for k in range(GROUP // LANES):
            dloc[i, pl.ds(k * LANES, LANES)] = zero16
        return 0
    lax.fori_loop(0, DR2 // GROUP, zbody, 0)

    rbase = s * (EPT_SCAT // GROUP)

    def ebody(g, _):
        pltpu.sync_copy(dst_hbm.at[pl.ds(rbase + g * 14, 14)], dbuf)
        for k in range(14):
            for t in range(GROUP // LANES):
                idx = dbuf[k, pl.ds(t * LANES, LANES)]
                plsc.addupdate_scatter(dloc, [idx >> 7, idx & 127], ones16)
        return 0
    lax.fori_loop(0, EPT_SCAT // GROUP // 14, ebody, 0)

    @pl.when(s == 0)
    def _():
        pltpu.sync_copy(dloc, spacc)
    for ci in range(4):
        for k in range(GROUP // LANES):
            ibuf[ci, pl.ds(k * LANES, LANES)] = iota16 + (ci * GROUP + k * LANES)
    plsc.subcore_barrier()

    @pl.when(s != 0)
    def _():
        for ci in range(4):
            pltpu.sync_copy(dloc.at[pl.ds(ci * GROUP, GROUP)],
                            spacc.at[ibuf.at[ci]], add=True)
    plsc.subcore_barrier()

    pltpu.sync_copy(spacc.at[pl.ds(16 * wid, 16)],
                    deg_hbm.at[pl.ds(16 * wid, 16)])


@functools.partial(
    pl.kernel,
    out_type=jax.ShapeDtypeStruct((NC * N, HH), f32),
    mesh=_mesh,
    scratch_types=[
        pltpu.VMEM_SHARED((ACC_ROWS, HH), f32),
        pltpu.VMEM((8, GROUP), i32),
        pltpu.VMEM((8, GROUP), i32),
        pltpu.VMEM((4, GROUP, HH), f32),
        pltpu.SemaphoreType.DMA((4,)),
        pltpu.SemaphoreType.DMA((4,)),
        pltpu.SemaphoreType.DMA((2,)),
    ],
    compiler_params=_sc_params,
)
def _sc_scatter(z_hbm, src_hbm, dst_hbm, acc_hbm,
                acc_sh, sbuf, dbuf, rows, gsems, ssems, isems):
    c = lax.axis_index("c")
    s = lax.axis_index("s")
    rbase = s * RPT
    zero16 = jnp.zeros((LANES,), f32)

    def zrow(i, _):
        rows[0, i, pl.ds(0, LANES)] = zero16
        rows[0, i, pl.ds(LANES, LANES)] = zero16
        return 0
    lax.fori_loop(0, GROUP, zrow, 0)

    nfull = RPT // GROUP
    rem = RPT - nfull * GROUP

    def zcp(i, _):
        pltpu.sync_copy(rows.at[0], acc_sh.at[pl.ds(rbase + i * GROUP, GROUP)])
        return 0
    lax.fori_loop(0, nfull, zcp, 0)
    pltpu.sync_copy(rows.at[0, pl.ds(0, rem)],
                    acc_sh.at[pl.ds(rbase + nfull * GROUP, rem)])
    plsc.subcore_barrier()

    offv = jnp.full((LANES,), c, i32)
    row0 = s * (EPT_SCAT // GROUP)

    def _drain(b):
        pltpu.make_async_copy(z_hbm.at[pl.ds(0, GROUP)], rows.at[b],
                              ssems.at[b]).wait()

    def gbody(g, _):
        r0 = row0 + g * 8
        ids = pltpu.async_copy(src_hbm.at[pl.ds(r0, 8)], sbuf, isems.at[0])
        idd = pltpu.async_copy(dst_hbm.at[pl.ds(r0, 8)], dbuf, isems.at[1])

        @pl.when(g > 0)
        def _():
            for b in range(4):
                _drain(b)

        ids.wait()
        idd.wait()
        for k in range(8):
            for t in range(GROUP // LANES):
                v = sbuf[k, pl.ds(t * LANES, LANES)]
                sbuf[k, pl.ds(t * LANES, LANES)] = v + v + offv
        gds = [pltpu.async_copy(z_hbm.at[sbuf.at[b]], rows.at[b], gsems.at[b])
               for b in range(4)]
        sds = []
        for b in range(4):
            gds[b].wait()
            sds.append(pltpu.async_copy(
                rows.at[b], acc_sh.at[dbuf.at[b]], ssems.at[b], add=True))
        gds2 = []
        for b in range(4):
            sds[b].wait()
            gds2.append(pltpu.async_copy(z_hbm.at[sbuf.at[4 + b]],
                                         rows.at[b], gsems.at[b]))
        for b in range(4):
            gds2[b].wait()
            pltpu.async_copy(
                rows.at[b], acc_sh.at[dbuf.at[4 + b]], ssems.at[b], add=True)
        return 0
    lax.fori_loop(0, EPT_SCAT // GROUP // 8, gbody, 0)

    for b in range(4):
        _drain(b)

    plsc.subcore_barrier()
    pltpu.sync_copy(acc_sh.at[pl.ds(rbase, RPT)],
                    acc_hbm.at[pl.ds(c * N + rbase, RPT)])


@functools.partial(
    pl.kernel,
    out_type=jax.ShapeDtypeStruct((LP,), f32),
    mesh=_mesh,
    scratch_types=[
        pltpu.VMEM((N,), f32),
        pltpu.VMEM((LPT,), i32),
        pltpu.VMEM((LPT,), i32),
        pltpu.VMEM((LPT,), f32),
    ],
    compiler_params=_sc_params,
)
def _sc_score(p_hbm, i0_hbm, i1_hbm, out_hbm, pbuf, i0, i1, ob):
    c = lax.axis_index("c")
    s = lax.axis_index("s")
    wid = c * NS + s
    base = wid * LPT
    pltpu.sync_copy(p_hbm, pbuf)
    pltpu.sync_copy(i0_hbm.at[pl.ds(base, LPT)], i0)
    pltpu.sync_copy(i1_hbm.at[pl.ds(base, LPT)], i1)

    def sbody(j, _):
        a = plsc.load_gather(pbuf, [i0[pl.ds(j * LANES, LANES)]])
        b = plsc.load_gather(pbuf, [i1[pl.ds(j * LANES, LANES)]])
        t = a * b
        ob[pl.ds(j * LANES, LANES)] = 1.0 / (1.0 + jnp.exp(-t))
        return 0
    lax.fori_loop(0, LPT // LANES, sbody, 0)

    pltpu.sync_copy(ob, out_hbm.at[pl.ds(base, LPT)])


def _dinv_block(deg_ref):
    return lax.rsqrt(deg_ref[...] + 1.0)


def _pack(hd, z_ref):
    h3 = jnp.reshape(hd, (BROW // 2, 2, H))
    z_ref[:, :H] = h3[:, 0, :]
    z_ref[:, H:] = h3[:, 1, :]


def _unpack(z_block):
    z3 = jnp.stack([z_block[:, :H], z_block[:, H:]], axis=1)
    return jnp.reshape(z3, (BROW, H))


def _enc_body(xd_ref, xp_ref, Wd_ref, bd_ref, Wp_ref, bp_ref, W0_ref,
              degp_ref, z_ref):
    is_d = pl.program_id(0) < ND // BROW
    x = jnp.where(is_d, xd_ref[...], xp_ref[...])
    W = jnp.where(is_d, Wd_ref[...], Wp_ref[...])
    b = jnp.where(is_d, bd_ref[...], bp_ref[...])
    x1 = jnp.maximum(x @ W + b, 0.0)
    h = x1 @ W0_ref[...]
    d = _dinv_block(degp_ref)
    _pack(h * d, z_ref)


def _layer_body(acc_ref, zin_ref, degp_ref, b_ref, W_ref, z_ref):
    d = _dinv_block(degp_ref)
    zprev = _unpack(zin_ref[...])
    x = (jnp.concatenate([acc_ref[0], acc_ref[1]], axis=1) + zprev) * d
    x = jnp.maximum(x + b_ref[...], 0.0)
    y = x @ W_ref[...]
    _pack(y * d, z_ref)


def _final_body(acc_ref, zin_ref, degp_ref, b_ref, lw_ref, lb_ref, p_ref):
    d = _dinv_block(degp_ref)
    zprev = _unpack(zin_ref[...])
    x = (jnp.concatenate([acc_ref[0], acc_ref[1]], axis=1) + zprev) * d
    x = jnp.maximum(x + b_ref[...], 0.0)
    p_ref[...] = x @ lw_ref[...] + lb_ref[0, 0]


def _full(shape):
    return pl.BlockSpec(shape, lambda i: tuple(0 for _ in shape))


_GRID = N // BROW

_enc_call = pl.pallas_call(
    _enc_body,
    grid=(_GRID,),
    in_specs=[
        pl.BlockSpec((BROW, DIN), lambda i: (jnp.minimum(i, ND // BROW - 1), 0)),
        pl.BlockSpec((BROW, DIN), lambda i: (jnp.maximum(i - ND // BROW, 0), 0)),
        _full((DIN, H)), _full((1, H)), _full((DIN, H)), _full((1, H)),
        _full((H, H)),
        pl.BlockSpec((BROW, 1), lambda i: (i, 0)),
    ],
    out_specs=pl.BlockSpec((BROW // 2, 2 * H), lambda i: (i, 0)),
    out_shape=jax.ShapeDtypeStruct((N // 2, 2 * H), f32),
)

_layer_call = pl.pallas_call(
    _layer_body,
    grid=(_GRID,),
    in_specs=[
        pl.BlockSpec((2, BROW, HH), lambda i: (0, i, 0)),
        pl.BlockSpec((BROW // 2, 2 * H), lambda i: (i, 0)),
        pl.BlockSpec((BROW, 1), lambda i: (i, 0)),
        _full((1, H)), _full((H, H)),
    ],
    out_specs=pl.BlockSpec((BROW // 2, 2 * H), lambda i: (i, 0)),
    out_shape=jax.ShapeDtypeStruct((N // 2, 2 * H), f32),
)

_final_call = pl.pallas_call(
    _final_body,
    grid=(_GRID,),
    in_specs=[
        pl.BlockSpec((2, BROW, HH), lambda i: (0, i, 0)),
        pl.BlockSpec((BROW // 2, 2 * H), lambda i: (i, 0)),
        pl.BlockSpec((BROW, 1), lambda i: (i, 0)),
        _full((1, H)), _full((H, 1)), _full((1, 1)),
    ],
    out_specs=pl.BlockSpec((BROW, 1), lambda i: (i, 0)),
    out_shape=jax.ShapeDtypeStruct((N, 1), f32),
)


def kernel(x_drug, x_protein, edge_index, edge_label_index,
           W_drug, b_drug, W_prot, b_prot,
           conv_W0, conv_b0, conv_W1, conv_b1, conv_W2, conv_b2,
           lin_W, lin_b):
    npad = EP - E
    src = jnp.concatenate(
        [edge_index[0], jnp.zeros((npad,), i32)]).reshape(EP // GROUP, GROUP)
    dst = jnp.concatenate(
        [edge_index[1], jnp.full((npad,), N, i32)]).reshape(EP // GROUP, GROUP)

    degp = _sc_degree(dst).reshape(DR2, 1)[:N]

    zP = _enc_call(x_drug, x_protein, W_drug, b_drug.reshape(1, H),
                   W_prot, b_prot.reshape(1, H), conv_W0, degp)
    for Wb in ((conv_b0, conv_W1), (conv_b1, conv_W2)):
        acc = _sc_scatter(zP.reshape(NC * N, HH), src, dst).reshape(2, N, HH)
        zP = _layer_call(acc, zP, degp, Wb[0].reshape(1, H), Wb[1])

    acc = _sc_scatter(zP.reshape(NC * N, HH), src, dst).reshape(2, N, HH)
    p = _final_call(acc, zP, degp, conv_b2.reshape(1, H),
                    lin_W, lin_b.reshape(1, 1)).reshape(N)

    lpad = LP - LBL
    i0 = jnp.concatenate([edge_label_index[0], jnp.zeros((lpad,), i32)])
    i1 = jnp.concatenate([edge_label_index[1], jnp.zeros((lpad,), i32)])
    return _sc_score(p, i0, i1)[:LBL]


# speedup vs baseline: 1.0934x; 1.0148x over previous
import functools

import jax
import jax.numpy as jnp
from jax import lax
from jax.experimental import pallas as pl
from jax.experimental.pallas import tpu as pltpu
from jax.experimental.pallas import tpu_sc as plsc

ND = 10000
NP = 40000
N = ND + NP
E = 800000
LBL = 100000
DIN = 128
H = 64
HH = 32

NC, NS, LANES = 2, 16, 16
EP = 802816
EPT_SCAT = EP // NS
EPT_DEG = EP // (NC * NS)
GROUP = 128
DR2 = 65536
ACC_ROWS = N + LANES
RPT = N // NS
LP = 100352
LPT = LP // (NC * NS)
BROW = 2000

_mesh = plsc.VectorSubcoreMesh(
    core_axis_name="c", subcore_axis_name="s", num_cores=NC, num_subcores=NS)
_sc_params = pltpu.CompilerParams(
    needs_layout_passes=False, use_tc_tiling_on_sc=False)

f32 = jnp.float32
i32 = jnp.int32


@functools.partial(
    pl.kernel,
    out_type=jax.ShapeDtypeStruct((2 * DR2 // GROUP, GROUP), f32),
    mesh=_mesh,
    scratch_types=[
        pltpu.VMEM((DR2 // GROUP, GROUP), f32),
        pltpu.VMEM((14, GROUP), i32),
        pltpu.VMEM((4, GROUP), i32),
        pltpu.VMEM_SHARED((DR2 // GROUP, GROUP), f32),
    ],
    compiler_params=_sc_params,
)
def _sc_degree(dst_hbm, deg_hbm, dloc, dbuf, ibuf, spacc):
    c = lax.axis_index("c")
    s = lax.axis_index("s")
    wid = c * NS + s
    zero16 = jnp.zeros((LANES,), f32)
    ones16 = jnp.ones((LANES,), f32)
    iota16 = jnp.arange(LANES, dtype=i32)

    def zbody(i, _):
        for k in range(GROUP // LANES):
            dloc[i, pl.ds(k * LANES, LANES)] = zero16
        return 0
    lax.fori_loop(0, DR2 // GROUP, zbody, 0)

    rbase = wid * (EPT_DEG // GROUP)

    def ebody(g, _):
        pltpu.sync_copy(dst_hbm.at[pl.ds(rbase + g * 14, 14)], dbuf)
        for k in range(14):
            for t in range(GROUP // LANES):
                idx = dbuf[k, pl.ds(t * LANES, LANES)]
                plsc.addupdate_scatter(dloc, [idx >> 7, idx & 127], ones16)
        return 0
    lax.fori_loop(0, EPT_DEG // GROUP // 14, ebody, 0)

    @pl.when(s == 0)
    def _():
        pltpu.sync_copy(dloc, spacc)
    for ci in range(4):
        for k in range(GROUP // LANES):
            ibuf[ci, pl.ds(k * LANES, LANES)] = iota16 + (ci * GROUP + k * LANES)
    plsc.subcore_barrier()

    @pl.when(s != 0)
    def _():
        for ci in range(4):
            pltpu.sync_copy(dloc.at[pl.ds(ci * GROUP, GROUP)],
                            spacc.at[ibuf.at[ci]], add=True)
    plsc.subcore_barrier()

    pltpu.sync_copy(spacc.at[pl.ds(32 * s, 32)],
                    deg_hbm.at[pl.ds(c * (DR2 // GROUP) + 32 * s, 32)])


@functools.partial(
    pl.kernel,
    out_type=jax.ShapeDtypeStruct((NC * N, HH), f32),
    mesh=_mesh,
    scratch_types=[
        pltpu.VMEM_SHARED((ACC_ROWS, HH), f32),
        pltpu.VMEM((8, GROUP), i32),
        pltpu.VMEM((8, GROUP), i32),
        pltpu.VMEM((4, GROUP, HH), f32),
        pltpu.SemaphoreType.DMA((4,)),
        pltpu.SemaphoreType.DMA((4,)),
        pltpu.SemaphoreType.DMA((2,)),
    ],
    compiler_params=_sc_params,
)
def _sc_scatter(z_hbm, src_hbm, dst_hbm, acc_hbm,
                acc_sh, sbuf, dbuf, rows, gsems, ssems, isems):
    c = lax.axis_index("c")
    s = lax.axis_index("s")
    rbase = s * RPT
    zero16 = jnp.zeros((LANES,), f32)

    def zrow(i, _):
        rows[0, i, pl.ds(0, LANES)] = zero16
        rows[0, i, pl.ds(LANES, LANES)] = zero16
        return 0
    lax.fori_loop(0, GROUP, zrow, 0)

    nfull = RPT // GROUP
    rem = RPT - nfull * GROUP

    def zcp(i, _):
        pltpu.sync_copy(rows.at[0], acc_sh.at[pl.ds(rbase + i * GROUP, GROUP)])
        return 0
    lax.fori_loop(0, nfull, zcp, 0)
    pltpu.sync_copy(rows.at[0, pl.ds(0, rem)],
                    acc_sh.at[pl.ds(rbase + nfull * GROUP, rem)])
    plsc.subcore_barrier()

    offv = jnp.full((LANES,), c, i32)
    row0 = s * (EPT_SCAT // GROUP)

    def _drain(b):
        pltpu.make_async_copy(z_hbm.at[pl.ds(0, GROUP)], rows.at[b],
                              ssems.at[b]).wait()

    def gbody(g, _):
        r0 = row0 + g * 8
        ids = pltpu.async_copy(src_hbm.at[pl.ds(r0, 8)], sbuf, isems.at[0])
        idd = pltpu.async_copy(dst_hbm.at[pl.ds(r0, 8)], dbuf, isems.at[1])

        @pl.when(g > 0)
        def _():
            for b in range(4):
                _drain(b)

        ids.wait()
        idd.wait()
        for k in range(8):
            for t in range(GROUP // LANES):
                v = sbuf[k, pl.ds(t * LANES, LANES)]
                sbuf[k, pl.ds(t * LANES, LANES)] = v + v + offv
        gds = [pltpu.async_copy(z_hbm.at[sbuf.at[b]], rows.at[b], gsems.at[b])
               for b in range(4)]
        sds = []
        for b in range(4):
            gds[b].wait()
            sds.append(pltpu.async_copy(
                rows.at[b], acc_sh.at[dbuf.at[b]], ssems.at[b], add=True))
        gds2 = []
        for b in range(4):
            sds[b].wait()
            gds2.append(pltpu.async_copy(z_hbm.at[sbuf.at[4 + b]],
                                         rows.at[b], gsems.at[b]))
        for b in range(4):
            gds2[b].wait()
            pltpu.async_copy(
                rows.at[b], acc_sh.at[dbuf.at[4 + b]], ssems.at[b], add=True)
        return 0
    lax.fori_loop(0, EPT_SCAT // GROUP // 8, gbody, 0)

    for b in range(4):
        _drain(b)

    plsc.subcore_barrier()
    pltpu.sync_copy(acc_sh.at[pl.ds(rbase, RPT)],
                    acc_hbm.at[pl.ds(c * N + rbase, RPT)])


@functools.partial(
    pl.kernel,
    out_type=jax.ShapeDtypeStruct((LP,), f32),
    mesh=_mesh,
    scratch_types=[
        pltpu.VMEM((N,), f32),
        pltpu.VMEM((LPT,), i32),
        pltpu.VMEM((LPT,), i32),
        pltpu.VMEM((LPT,), f32),
    ],
    compiler_params=_sc_params,
)
def _sc_score(p_hbm, i0_hbm, i1_hbm, out_hbm, pbuf, i0, i1, ob):
    c = lax.axis_index("c")
    s = lax.axis_index("s")
    wid = c * NS + s
    base = wid * LPT
    pltpu.sync_copy(p_hbm, pbuf)
    pltpu.sync_copy(i0_hbm.at[pl.ds(base, LPT)], i0)
    pltpu.sync_copy(i1_hbm.at[pl.ds(base, LPT)], i1)

    def sbody(j, _):
        a = plsc.load_gather(pbuf, [i0[pl.ds(j * LANES, LANES)]])
        b = plsc.load_gather(pbuf, [i1[pl.ds(j * LANES, LANES)]])
        t = a * b
        ob[pl.ds(j * LANES, LANES)] = 1.0 / (1.0 + jnp.exp(-t))
        return 0
    lax.fori_loop(0, LPT // LANES, sbody, 0)

    pltpu.sync_copy(ob, out_hbm.at[pl.ds(base, LPT)])


def _dinv_block(deg_ref):
    return lax.rsqrt(deg_ref[...] + 1.0)


def _pack(hd, z_ref):
    h3 = jnp.reshape(hd, (BROW // 2, 2, H))
    z_ref[:, :H] = h3[:, 0, :]
    z_ref[:, H:] = h3[:, 1, :]


def _unpack(z_block):
    z3 = jnp.stack([z_block[:, :H], z_block[:, H:]], axis=1)
    return jnp.reshape(z3, (BROW, H))


def _enc_body(xd_ref, xp_ref, Wd_ref, bd_ref, Wp_ref, bp_ref, W0_ref,
              degp_ref, z_ref):
    is_d = pl.program_id(0) < ND // BROW
    x = jnp.where(is_d, xd_ref[...], xp_ref[...])
    W = jnp.where(is_d, Wd_ref[...], Wp_ref[...])
    b = jnp.where(is_d, bd_ref[...], bp_ref[...])
    x1 = jnp.maximum(x @ W + b, 0.0)
    h = x1 @ W0_ref[...]
    d = _dinv_block(degp_ref)
    _pack(h * d, z_ref)


def _layer_body(acc_ref, zin_ref, degp_ref, b_ref, W_ref, z_ref):
    d = _dinv_block(degp_ref)
    zprev = _unpack(zin_ref[...])
    x = (jnp.concatenate([acc_ref[0], acc_ref[1]], axis=1) + zprev) * d
    x = jnp.maximum(x + b_ref[...], 0.0)
    y = x @ W_ref[...]
    _pack(y * d, z_ref)


def _final_body(acc_ref, zin_ref, degp_ref, b_ref, lw_ref, lb_ref, p_ref):
    d = _dinv_block(degp_ref)
    zprev = _unpack(zin_ref[...])
    x = (jnp.concatenate([acc_ref[0], acc_ref[1]], axis=1) + zprev) * d
    x = jnp.maximum(x + b_ref[...], 0.0)
    p_ref[...] = x @ lw_ref[...] + lb_ref[0, 0]


def _full(shape):
    return pl.BlockSpec(shape, lambda i: tuple(0 for _ in shape))


_GRID = N // BROW

_enc_call = pl.pallas_call(
    _enc_body,
    grid=(_GRID,),
    in_specs=[
        pl.BlockSpec((BROW, DIN), lambda i: (jnp.minimum(i, ND // BROW - 1), 0)),
        pl.BlockSpec((BROW, DIN), lambda i: (jnp.maximum(i - ND // BROW, 0), 0)),
        _full((DIN, H)), _full((1, H)), _full((DIN, H)), _full((1, H)),
        _full((H, H)),
        pl.BlockSpec((BROW, 1), lambda i: (i, 0)),
    ],
    out_specs=pl.BlockSpec((BROW // 2, 2 * H), lambda i: (i, 0)),
    out_shape=jax.ShapeDtypeStruct((N // 2, 2 * H), f32),
)

_layer_call = pl.pallas_call(
    _layer_body,
    grid=(_GRID,),
    in_specs=[
        pl.BlockSpec((2, BROW, HH), lambda i: (0, i, 0)),
        pl.BlockSpec((BROW // 2, 2 * H), lambda i: (i, 0)),
        pl.BlockSpec((BROW, 1), lambda i: (i, 0)),
        _full((1, H)), _full((H, H)),
    ],
    out_specs=pl.BlockSpec((BROW // 2, 2 * H), lambda i: (i, 0)),
    out_shape=jax.ShapeDtypeStruct((N // 2, 2 * H), f32),
)

_final_call = pl.pallas_call(
    _final_body,
    grid=(_GRID,),
    in_specs=[
        pl.BlockSpec((2, BROW, HH), lambda i: (0, i, 0)),
        pl.BlockSpec((BROW // 2, 2 * H), lambda i: (i, 0)),
        pl.BlockSpec((BROW, 1), lambda i: (i, 0)),
        _full((1, H)), _full((H, 1)), _full((1, 1)),
    ],
    out_specs=pl.BlockSpec((BROW, 1), lambda i: (i, 0)),
    out_shape=jax.ShapeDtypeStruct((N, 1), f32),
)


def kernel(x_drug, x_protein, edge_index, edge_label_index,
           W_drug, b_drug, W_prot, b_prot,
           conv_W0, conv_b0, conv_W1, conv_b1, conv_W2, conv_b2,
           lin_W, lin_b):
    npad = EP - E
    src = jnp.concatenate(
        [edge_index[0], jnp.zeros((npad,), i32)]).reshape(EP // GROUP, GROUP)
    dst = jnp.concatenate(
        [edge_index[1], jnp.full((npad,), N, i32)]).reshape(EP // GROUP, GROUP)

    dpart = _sc_degree(dst).reshape(2, DR2, 1)
    degp = (dpart[0] + dpart[1])[:N]

    zP = _enc_call(x_drug, x_protein, W_drug, b_drug.reshape(1, H),
                   W_prot, b_prot.reshape(1, H), conv_W0, degp)
    for Wb in ((conv_b0, conv_W1), (conv_b1, conv_W2)):
        acc = _sc_scatter(zP.reshape(NC * N, HH), src, dst).reshape(2, N, HH)
        zP = _layer_call(acc, zP, degp, Wb[0].reshape(1, H), Wb[1])

    acc = _sc_scatter(zP.reshape(NC * N, HH), src, dst).reshape(2, N, HH)
    p = _final_call(acc, zP, degp, conv_b2.reshape(1, H),
                    lin_W, lin_b.reshape(1, 1)).reshape(N)

    lpad = LP - LBL
    i0 = jnp.concatenate([edge_label_index[0], jnp.zeros((lpad,), i32)])
    i1 = jnp.concatenate([edge_label_index[1], jnp.zeros((lpad,), i32)])
    return _sc_score(p, i0, i1)[:LBL]
